# baseline (speedup 1.0000x reference)
import functools

import jax
import jax.numpy as jnp
from jax import lax
from jax.experimental import pallas as pl
from jax.experimental.pallas import tpu as pltpu
from jax.experimental.pallas import tpu_sc as plsc

EMBED_DIM = 16
NIDX = 8


@functools.partial(
    pl.kernel,
    out_type=jax.ShapeDtypeStruct((EMBED_DIM,), jnp.float32),
    mesh=plsc.VectorSubcoreMesh(core_axis_name="c", subcore_axis_name="s"),
    scratch_types=[
        pltpu.VMEM((NIDX,), jnp.int32),
        pltpu.VMEM((NIDX, EMBED_DIM), jnp.float32),
        pltpu.SemaphoreType.DMA,
    ],
    compiler_params=pltpu.CompilerParams(use_tc_tiling_on_sc=False),
)
def _row_gather(table_hbm, ids_hbm, out_hbm, idx_v, rows_v, sem):
    cid = lax.axis_index("c")
    sid = lax.axis_index("s")

    @pl.when(jnp.logical_and(cid == 0, sid == 0))
    def _():
        pltpu.sync_copy(ids_hbm.at[pl.ds(0, NIDX)], idx_v)
        pltpu.async_copy(table_hbm.at[idx_v], rows_v, sem).wait()
        pltpu.sync_copy(rows_v.at[0], out_hbm)


def kernel(ids, table):
    ids_flat = ids.reshape(-1).astype(jnp.int32)
    return _row_gather(table, ids_flat)


# speedup vs baseline: 1.6629x; 1.6629x over previous
import functools

import jax
import jax.numpy as jnp
from jax.experimental import pallas as pl
from jax.experimental.pallas import tpu as pltpu
from jax.experimental.pallas import tpu_sc as plsc

EMBED_DIM = 16
NIDX = 16
ROWS_PER_TILE = 8


@functools.partial(
    pl.kernel,
    out_type=jax.ShapeDtypeStruct((EMBED_DIM,), jnp.float32),
    mesh=plsc.VectorSubcoreMesh(
        core_axis_name="c", subcore_axis_name="s", num_cores=1, num_subcores=1
    ),
    scratch_types=[
        pltpu.VMEM((NIDX,), jnp.int32),
        pltpu.VMEM((ROWS_PER_TILE, EMBED_DIM), jnp.float32),
        pltpu.VMEM((EMBED_DIM,), jnp.float32),
    ],
)
def _row_gather(table_hbm, ids_hbm, out_hbm, idx_v, tile_v, row_v):
    pltpu.sync_copy(ids_hbm.at[pl.ds(0, NIDX)], idx_v)
    idx = idx_v[...][0]
    base = (idx // ROWS_PER_TILE) * ROWS_PER_TILE
    pltpu.sync_copy(table_hbm.at[pl.ds(base, ROWS_PER_TILE)], tile_v)
    sub = idx - base
    row = tile_v[0, :]
    for j in range(1, ROWS_PER_TILE):
        row = jnp.where(sub == j, tile_v[j, :], row)
    row_v[...] = row
    pltpu.sync_copy(row_v, out_hbm)


def kernel(ids, table):
    ids_flat = ids.reshape(-1).astype(jnp.int32)
    return _row_gather(table, ids_flat)


# speedup vs baseline: 1.6748x; 1.0072x over previous
import functools

import jax
import jax.numpy as jnp
from jax.experimental import pallas as pl
from jax.experimental.pallas import tpu as pltpu
from jax.experimental.pallas import tpu_sc as plsc

EMBED_DIM = 16
NIDX = 16
ROWS_PER_TILE = 8


@functools.partial(
    pl.kernel,
    out_type=jax.ShapeDtypeStruct((EMBED_DIM,), jnp.float32),
    mesh=plsc.VectorSubcoreMesh(
        core_axis_name="c", subcore_axis_name="s", num_cores=1, num_subcores=1
    ),
    scratch_types=[
        pltpu.VMEM((NIDX,), jnp.int32),
        pltpu.VMEM((ROWS_PER_TILE, EMBED_DIM), jnp.float32),
        pltpu.VMEM((EMBED_DIM,), jnp.float32),
    ],
)
def _row_gather(table_hbm, ids_hbm, out_hbm, idx_v, tile_v, row_v):
    pltpu.sync_copy(ids_hbm.at[0, pl.ds(0, NIDX)], idx_v)
    idx = idx_v[...][0]
    base = (idx // ROWS_PER_TILE) * ROWS_PER_TILE
    pltpu.sync_copy(table_hbm.at[pl.ds(base, ROWS_PER_TILE)], tile_v)
    sub = idx - base
    row = tile_v[0, :]
    for j in range(1, ROWS_PER_TILE):
        row = jnp.where(sub == j, tile_v[j, :], row)
    row_v[...] = row
    pltpu.sync_copy(row_v, out_hbm)


def kernel(ids, table):
    return _row_gather(table, ids.astype(jnp.int32))


# speedup vs baseline: 1.6804x; 1.0034x over previous
import functools

import jax
import jax.numpy as jnp
from jax.experimental import pallas as pl
from jax.experimental.pallas import tpu as pltpu
from jax.experimental.pallas import tpu_sc as plsc

EMBED_DIM = 16
NIDX = 16
ROWS_PER_TILE = 8


@functools.partial(
    pl.kernel,
    out_type=jax.ShapeDtypeStruct((EMBED_DIM,), jnp.float32),
    mesh=plsc.VectorSubcoreMesh(
        core_axis_name="c", subcore_axis_name="s", num_cores=1, num_subcores=1
    ),
    scratch_types=[
        pltpu.VMEM((NIDX,), jnp.int32),
        pltpu.VMEM((ROWS_PER_TILE, EMBED_DIM), jnp.float32),
        pltpu.VMEM((EMBED_DIM,), jnp.float32),
    ],
)
def _row_gather(table_hbm, ids_hbm, out_hbm, idx_v, tile_v, row_v):
    pltpu.sync_copy(ids_hbm.at[0, pl.ds(0, NIDX)], idx_v)
    idx = idx_v[...][0]
    base = (idx // ROWS_PER_TILE) * ROWS_PER_TILE
    pltpu.sync_copy(table_hbm.at[pl.ds(base, ROWS_PER_TILE)], tile_v)
    sub = idx - base
    row = tile_v[0, :]
    for j in range(1, ROWS_PER_TILE):
        row = jnp.where(sub == j, tile_v[j, :], row)
    row_v[...] = row
    pltpu.sync_copy(row_v, out_hbm)


def kernel(ids, table):
    ids_head = jax.lax.slice(ids, (0, 0), (1, NIDX)).astype(jnp.int32)
    return _row_gather(table, ids_head)
